# baseline (speedup 1.0000x reference)
import functools
import math

import jax
import jax.numpy as jnp
from jax import lax
from jax.experimental import pallas as pl
from jax.experimental.pallas import tpu as pltpu
from jax.experimental.pallas import tpu_sc as plsc

DMODEL = 64
SCALE = math.sqrt(DMODEL)
NROWS, NCOLS = 4096, 200
NC, NS = 2, 16
NW = NC * NS
BLK = 512
SUB = 128
NSUB = BLK // SUB
NBLK_I = NROWS // BLK
TOTAL_BLK = NCOLS * NBLK_I
BPW = TOTAL_BLK // NW
NBUF = 2
LANES = 16


def _emb_body(x_hbm, table_hbm, out_hbm, idx, rows, tbuf, sems):
    wid = lax.axis_index("s") * NC + lax.axis_index("c")
    g0 = wid * BPW

    def stage(g, b):
        j = g // NBLK_I
        ib = g % NBLK_I
        pltpu.sync_copy(x_hbm.at[j, pl.ds(ib * NSUB, NSUB)], idx[b])
        for k in range(NSUB):
            pltpu.async_copy(
                table_hbm.at[idx[b].at[k]],
                rows[b].at[pl.ds(k * SUB, SUB)],
                sems[b],
            )

    def wait_gathers(b):
        for k in range(NSUB):
            pltpu.make_async_copy(
                table_hbm.at[idx[b].at[k]],
                rows[b].at[pl.ds(k * SUB, SUB)],
                sems[b],
            ).wait()

    iota = lax.iota(jnp.int32, LANES)

    def process(g, b):
        @pl.loop(0, DMODEL)
        def _(d):
            dcol = jnp.full((LANES,), 0, jnp.int32) + d
            @pl.loop(0, BLK // LANES)
            def _(t):
                ridx = iota + t * LANES
                v = plsc.load_gather(rows[b], [ridx, dcol])
                tbuf[d, pl.ds(t * LANES, LANES)] = v * SCALE
        j = g // NBLK_I
        i0 = (g % NBLK_I) * BLK
        pltpu.sync_copy(tbuf, out_hbm.at[j, :, pl.ds(i0, BLK)])

    for b in range(NBUF):
        stage(g0 + b, b)

    @pl.loop(0, BPW - NBUF, step=NBUF)
    def _(lg):
        for b in range(NBUF):
            g = g0 + lg + b
            wait_gathers(b)
            process(g, b)
            stage(g + NBUF, b)

    for b in range(NBUF):
        g = g0 + BPW - NBUF + b
        wait_gathers(b)
        process(g, b)


@functools.partial(
    pl.kernel,
    out_type=jax.ShapeDtypeStruct((NCOLS, DMODEL, NROWS), jnp.float32),
    mesh=plsc.VectorSubcoreMesh(core_axis_name="c", subcore_axis_name="s"),
    scratch_types=dict(
        idx=[pltpu.VMEM((NSUB, SUB), jnp.int32) for _ in range(NBUF)],
        rows=[pltpu.VMEM((BLK, DMODEL), jnp.float32) for _ in range(NBUF)],
        tbuf=pltpu.VMEM((DMODEL, BLK), jnp.float32),
        sems=[pltpu.SemaphoreType.DMA for _ in range(NBUF)],
    ),
    compiler_params=pltpu.CompilerParams(
        use_tc_tiling_on_sc=False, needs_layout_passes=False
    ),
)
def _emb(x_hbm, table_hbm, out_hbm, idx, rows, tbuf, sems):
    _emb_body(x_hbm, table_hbm, out_hbm, idx, rows, tbuf, sems)


def kernel(x, table):
    xt = x.astype(jnp.int32).T.reshape(NCOLS, NROWS // SUB, SUB)
    outp = _emb(xt, table)
    return outp.transpose(2, 0, 1)


# speedup vs baseline: 1.1106x; 1.1106x over previous
import functools
import math

import jax
import jax.numpy as jnp
from jax import lax
from jax.experimental import pallas as pl
from jax.experimental.pallas import tpu as pltpu
from jax.experimental.pallas import tpu_sc as plsc

DMODEL = 64
SCALE = math.sqrt(DMODEL)
NROWS, NCOLS = 4096, 200
NC, NS = 2, 16
NW = NC * NS
BLK = 512
SUB = 128
NSUB = BLK // SUB
NBLK_I = NROWS // BLK
TOTAL_BLK = NCOLS * NBLK_I
BPW = TOTAL_BLK // NW
NBUF = 2
LANES = 16
PITCH = DMODEL
TI = NROWS // SUB
TJ = NCOLS // 8


def _emb_body(x_hbm, table_hbm, out_hbm, idx, rows, tbuf, sems):
    wid = lax.axis_index("s") * NC + lax.axis_index("c")
    g0 = wid * BPW
    iota = lax.iota(jnp.int32, LANES)

    def stage(g, b):
        j = g // NBLK_I
        tc0 = (g % NBLK_I) * NSUB
        pltpu.sync_copy(
            x_hbm.at[j // 8, pl.ds(tc0, NSUB), j % 8, :], idx[b]
        )
        for k in range(NSUB):
            pltpu.async_copy(
                table_hbm.at[idx[b].at[k]],
                rows[b].at[pl.ds(k * SUB, SUB)],
                sems[b],
            )

    def wait_gathers(b):
        for k in range(NSUB):
            pltpu.make_async_copy(
                table_hbm.at[idx[b].at[k]],
                rows[b].at[pl.ds(k * SUB, SUB)],
                sems[b],
            ).wait()

    def process(g, b):
        @pl.loop(0, DMODEL)
        def _(d):
            dcol = jnp.full((LANES,), 0, jnp.int32) + d
            dt = d // 8
            dr = d % 8
            for it in range(NSUB):
                for t in range(SUB // LANES):
                    ridx = iota + (it * SUB + t * LANES)
                    v = plsc.load_gather(rows[b], [ridx, dcol])
                    tbuf[dt, it, dr, pl.ds(t * LANES, LANES)] = v * SCALE
        j = g // NBLK_I
        tc0 = (g % NBLK_I) * NSUB
        pltpu.sync_copy(tbuf, out_hbm.at[j, :, pl.ds(tc0, NSUB), :, :])

    for b in range(NBUF):
        stage(g0 + b, b)

    @pl.loop(0, BPW - NBUF, step=NBUF)
    def _(lg):
        for b in range(NBUF):
            g = g0 + lg + b
            wait_gathers(b)
            process(g, b)
            stage(g + NBUF, b)

    for b in range(NBUF):
        g = g0 + BPW - NBUF + b
        wait_gathers(b)
        process(g, b)


@functools.partial(
    pl.kernel,
    out_type=jax.ShapeDtypeStruct((NCOLS, 8, TI, 8, SUB), jnp.float32),
    mesh=plsc.VectorSubcoreMesh(core_axis_name="c", subcore_axis_name="s"),
    scratch_types=dict(
        idx=[pltpu.VMEM((NSUB, SUB), jnp.int32) for _ in range(NBUF)],
        rows=[pltpu.VMEM((BLK, PITCH), jnp.float32) for _ in range(NBUF)],
        tbuf=pltpu.VMEM((8, NSUB, 8, SUB), jnp.float32),
        sems=[pltpu.SemaphoreType.DMA for _ in range(NBUF)],
    ),
    compiler_params=pltpu.CompilerParams(
        use_tc_tiling_on_sc=False, needs_layout_passes=False
    ),
)
def _emb(x_hbm, table_hbm, out_hbm, idx, rows, tbuf, sems):
    _emb_body(x_hbm, table_hbm, out_hbm, idx, rows, tbuf, sems)


def kernel(x, table):
    x4 = (
        x.astype(jnp.int32)
        .T.reshape(TJ, 8, TI, SUB)
        .transpose(0, 2, 1, 3)
    )
    out5 = _emb(x4, table)
    return out5.transpose(2, 4, 0, 1, 3).reshape(NROWS, NCOLS, DMODEL)


# speedup vs baseline: 1.6705x; 1.5041x over previous
import functools
import math

import jax
import jax.numpy as jnp
from jax import lax
from jax.experimental import pallas as pl
from jax.experimental.pallas import tpu as pltpu
from jax.experimental.pallas import tpu_sc as plsc

DMODEL = 64
SCALE = math.sqrt(DMODEL)
NROWS, NCOLS = 4096, 200
NC, NS = 2, 16
NW = NC * NS
BLK = 512
SUB = 128
NSUB = BLK // SUB
NBLK_I = NROWS // BLK
TOTAL_BLK = NCOLS * NBLK_I
BPW = TOTAL_BLK // NW
NBUF = 2
LANES = 16
PITCH = DMODEL
TI = NROWS // SUB
TJ = NCOLS // 8


def _emb_body(x_hbm, table_hbm, out_hbm, idx, rows, tbuf, sems):
    wid = lax.axis_index("s") * NC + lax.axis_index("c")
    g0 = wid * BPW
    iota = lax.iota(jnp.int32, LANES)

    def stage(g, b):
        j = g // NBLK_I
        tc0 = (g % NBLK_I) * NSUB
        pltpu.sync_copy(
            x_hbm.at[j // 8, pl.ds(tc0, NSUB), j % 8, :], idx[b]
        )
        for k in range(NSUB):
            pltpu.async_copy(
                table_hbm.at[idx[b].at[k]],
                rows[b].at[pl.ds(k * SUB, SUB)],
                sems[b],
            )

    def wait_gathers(b):
        for k in range(NSUB):
            pltpu.make_async_copy(
                table_hbm.at[idx[b].at[k]],
                rows[b].at[pl.ds(k * SUB, SUB)],
                sems[b],
            ).wait()

    def process(g, b):
        @plsc.parallel_loop(0, DMODEL, unroll=2)
        def _(d):
            dcol = jnp.full((LANES,), 0, jnp.int32) + d
            dt = d // 8
            dr = d % 8
            for it in range(NSUB):
                for t in range(SUB // LANES):
                    ridx = iota + (it * SUB + t * LANES)
                    v = plsc.load_gather(rows[b], [ridx, dcol])
                    tbuf[dt, it, dr, pl.ds(t * LANES, LANES)] = v * SCALE
        j = g // NBLK_I
        tc0 = (g % NBLK_I) * NSUB
        pltpu.sync_copy(tbuf, out_hbm.at[j, :, pl.ds(tc0, NSUB), :, :])

    for b in range(NBUF):
        stage(g0 + b, b)

    @pl.loop(0, BPW - NBUF, step=NBUF)
    def _(lg):
        for b in range(NBUF):
            g = g0 + lg + b
            wait_gathers(b)
            process(g, b)
            stage(g + NBUF, b)

    for b in range(NBUF):
        g = g0 + BPW - NBUF + b
        wait_gathers(b)
        process(g, b)


@functools.partial(
    pl.kernel,
    out_type=jax.ShapeDtypeStruct((NCOLS, 8, TI, 8, SUB), jnp.float32),
    mesh=plsc.VectorSubcoreMesh(core_axis_name="c", subcore_axis_name="s"),
    scratch_types=dict(
        idx=[pltpu.VMEM((NSUB, SUB), jnp.int32) for _ in range(NBUF)],
        rows=[pltpu.VMEM((BLK, PITCH), jnp.float32) for _ in range(NBUF)],
        tbuf=pltpu.VMEM((8, NSUB, 8, SUB), jnp.float32),
        sems=[pltpu.SemaphoreType.DMA for _ in range(NBUF)],
    ),
    compiler_params=pltpu.CompilerParams(
        use_tc_tiling_on_sc=False, needs_layout_passes=False
    ),
)
def _emb(x_hbm, table_hbm, out_hbm, idx, rows, tbuf, sems):
    _emb_body(x_hbm, table_hbm, out_hbm, idx, rows, tbuf, sems)


def kernel(x, table):
    x4 = (
        x.astype(jnp.int32)
        .T.reshape(TJ, 8, TI, SUB)
        .transpose(0, 2, 1, 3)
    )
    out5 = _emb(x4, table)
    return out5.transpose(2, 4, 0, 1, 3).reshape(NROWS, NCOLS, DMODEL)


# speedup vs baseline: 2.4934x; 1.4926x over previous
import functools
import math

import jax
import jax.numpy as jnp
from jax import lax
from jax.experimental import pallas as pl
from jax.experimental.pallas import tpu as pltpu
from jax.experimental.pallas import tpu_sc as plsc

DMODEL = 64
PW = 128
SCALE = math.sqrt(DMODEL)
NROWS, NCOLS = 4096, 200
NC, NS = 2, 16
NW = NC * NS
BLK = 256
SUB = 128
NSUB = BLK // SUB
NBLK_I = NROWS // BLK
TOTAL_BLK = NCOLS * NBLK_I
BPW = TOTAL_BLK // NW
NBUF = 2
LANES = 16
TI = NROWS // SUB
TJ = NCOLS // 8
TBLK = BLK * DMODEL


def _emb_body(x_hbm, table_hbm, out_hbm, idx, rows, tbuf, sems):
    wid = lax.axis_index("s") * NC + lax.axis_index("c")
    g0 = wid * BPW
    iota = lax.iota(jnp.int32, LANES)

    def stage(g, b):
        j = g // NBLK_I
        tc0 = (g % NBLK_I) * NSUB
        pltpu.sync_copy(
            x_hbm.at[j // 8, pl.ds(tc0, NSUB), j % 8, :], idx[b]
        )
        for k in range(NSUB):
            pltpu.async_copy(
                table_hbm.at[idx[b].at[k]],
                rows[b].at[pl.ds(k * SUB, SUB)],
                sems[b],
            )

    def wait_gathers(b):
        for k in range(NSUB):
            pltpu.make_async_copy(
                table_hbm.at[idx[b].at[k]],
                rows[b].at[pl.ds(k * SUB, SUB)],
                sems[b],
            ).wait()

    def process(g, b):
        for s in range(LANES):
            m = (iota + s) & (LANES - 1)
            mhi = m >> 3
            mlo = m & 7
            for it in range(NSUB):
                itv = jnp.full((LANES,), it, jnp.int32)

                @plsc.parallel_loop(0, 32, unroll=2)
                def _(u):
                    a = u >> 2
                    q = u & 3
                    rvec = (it * SUB + a * LANES) + iota
                    dvec = (q * LANES) + m
                    dtvec = (q * 2) + mhi
                    icvec = (a * LANES) + iota
                    v = plsc.load_gather(rows[b], [rvec, dvec])
                    plsc.store_scatter(tbuf, [dtvec, itv, mlo, icvec], v * SCALE)

        j = g // NBLK_I
        tc0 = (g % NBLK_I) * NSUB
        pltpu.sync_copy(tbuf, out_hbm.at[j, :, pl.ds(tc0, NSUB), :, :])

    for b in range(NBUF):
        stage(g0 + b, b)

    @pl.loop(0, BPW - NBUF, step=NBUF)
    def _(lg):
        for b in range(NBUF):
            g = g0 + lg + b
            wait_gathers(b)
            process(g, b)
            stage(g + NBUF, b)

    for b in range(NBUF):
        g = g0 + BPW - NBUF + b
        wait_gathers(b)
        process(g, b)


@functools.partial(
    pl.kernel,
    out_type=jax.ShapeDtypeStruct((NCOLS, 8, TI, 8, SUB), jnp.float32),
    mesh=plsc.VectorSubcoreMesh(core_axis_name="c", subcore_axis_name="s"),
    scratch_types=dict(
        idx=[pltpu.VMEM((NSUB, SUB), jnp.int32) for _ in range(NBUF)],
        rows=[pltpu.VMEM((BLK, PW), jnp.float32) for _ in range(NBUF)],
        tbuf=pltpu.VMEM((8, NSUB, 8, SUB), jnp.float32),
        sems=[pltpu.SemaphoreType.DMA for _ in range(NBUF)],
    ),
    compiler_params=pltpu.CompilerParams(
        use_tc_tiling_on_sc=False, needs_layout_passes=False
    ),
)
def _emb(x_hbm, table_hbm, out_hbm, idx, rows, tbuf, sems):
    _emb_body(x_hbm, table_hbm, out_hbm, idx, rows, tbuf, sems)


def kernel(x, table):
    x4 = (
        x.astype(jnp.int32)
        .T.reshape(TJ, 8, TI, SUB)
        .transpose(0, 2, 1, 3)
    )
    tp = jnp.pad(table, ((0, 0), (0, PW - DMODEL)))
    out5 = _emb(x4, tp)
    return out5.transpose(2, 4, 0, 1, 3).reshape(NROWS, NCOLS, DMODEL)


# speedup vs baseline: 2.6183x; 1.0501x over previous
import functools
import math

import jax
import jax.numpy as jnp
from jax import lax
from jax.experimental import pallas as pl
from jax.experimental.pallas import tpu as pltpu
from jax.experimental.pallas import tpu_sc as plsc

DMODEL = 64
VOCAB = 1000000
PW = 128
SCALE = math.sqrt(DMODEL)
NROWS, NCOLS = 4096, 200
NC, NS = 2, 16
NW = NC * NS
BLK = 256
SUB = 128
NSUB = BLK // SUB
NBLK_I = NROWS // BLK
TOTAL_BLK = NCOLS * NBLK_I
BPW = TOTAL_BLK // NW
NBUF = 2
LANES = 16
TI = NROWS // SUB
TJ = NCOLS // 8
TBLK = BLK * DMODEL


PREP_C = 2048
VPAIR = ((VOCAB + PREP_C - 1) // PREP_C) * (PREP_C // 2)


def _prep_body(t_ref, o_ref):
    xt = t_ref[...].T
    h = PREP_C // 2
    o_ref[...] = jnp.concatenate([xt[:h, :], xt[h:, :]], axis=1)


_prep = pl.pallas_call(
    _prep_body,
    grid=((VOCAB + PREP_C - 1) // PREP_C,),
    in_specs=[pl.BlockSpec((DMODEL, PREP_C), lambda p: (0, p))],
    out_specs=pl.BlockSpec((PREP_C // 2, PW), lambda p: (p, 0)),
    out_shape=jax.ShapeDtypeStruct((VPAIR, PW), jnp.float32),
)


def _emb_body(x_hbm, table_hbm, out_hbm, idx, idx2, rows, tbuf, sems):
    wid = lax.axis_index("s") * NC + lax.axis_index("c")
    g0 = wid * BPW
    iota = lax.iota(jnp.int32, LANES)

    def stage(g, b):
        j = g // NBLK_I
        tc0 = (g % NBLK_I) * NSUB
        pltpu.sync_copy(
            x_hbm.at[j // 8, pl.ds(tc0, NSUB), j % 8, :], idx[b]
        )
        for k in range(NSUB):
            for t in range(SUB // LANES):
                sl = pl.ds(t * LANES, LANES)
                r = idx[b][k, sl]
                idx2[b][k, sl] = ((r >> 11) << 10) + (r & 1023)
        for k in range(NSUB):
            pltpu.async_copy(
                table_hbm.at[idx2[b].at[k]],
                rows[b].at[pl.ds(k * SUB, SUB)],
                sems[b],
            )

    def wait_gathers(b):
        for k in range(NSUB):
            pltpu.make_async_copy(
                table_hbm.at[idx2[b].at[k]],
                rows[b].at[pl.ds(k * SUB, SUB)],
                sems[b],
            ).wait()

    def process(g, b):
        for s in range(LANES):
            m = (iota + s) & (LANES - 1)
            mhi = m >> 3
            mlo = m & 7
            for it in range(NSUB):
                itv = jnp.full((LANES,), it, jnp.int32)

                @plsc.parallel_loop(0, 32, unroll=2)
                def _(u):
                    a = u >> 2
                    q = u & 3
                    hv = ((idx[b][it, pl.ds(a * LANES, LANES)] >> 10) & 1) << 6
                    rvec = (it * SUB + a * LANES) + iota
                    dvec = ((q * LANES) + m) + hv
                    dtvec = (q * 2) + mhi
                    icvec = (a * LANES) + iota
                    v = plsc.load_gather(rows[b], [rvec, dvec])
                    plsc.store_scatter(tbuf, [dtvec, itv, mlo, icvec], v * SCALE)

        j = g // NBLK_I
        tc0 = (g % NBLK_I) * NSUB
        pltpu.sync_copy(tbuf, out_hbm.at[j, :, pl.ds(tc0, NSUB), :, :])

    for b in range(NBUF):
        stage(g0 + b, b)

    @pl.loop(0, BPW - NBUF, step=NBUF)
    def _(lg):
        for b in range(NBUF):
            g = g0 + lg + b
            wait_gathers(b)
            process(g, b)
            stage(g + NBUF, b)

    for b in range(NBUF):
        g = g0 + BPW - NBUF + b
        wait_gathers(b)
        process(g, b)


@functools.partial(
    pl.kernel,
    out_type=jax.ShapeDtypeStruct((NCOLS, 8, TI, 8, SUB), jnp.float32),
    mesh=plsc.VectorSubcoreMesh(core_axis_name="c", subcore_axis_name="s"),
    scratch_types=dict(
        idx=[pltpu.VMEM((NSUB, SUB), jnp.int32) for _ in range(NBUF)],
        idx2=[pltpu.VMEM((NSUB, SUB), jnp.int32) for _ in range(NBUF)],
        rows=[pltpu.VMEM((BLK, PW), jnp.float32) for _ in range(NBUF)],
        tbuf=pltpu.VMEM((8, NSUB, 8, SUB), jnp.float32),
        sems=[pltpu.SemaphoreType.DMA for _ in range(NBUF)],
    ),
    compiler_params=pltpu.CompilerParams(
        use_tc_tiling_on_sc=False, needs_layout_passes=False
    ),
)
def _emb(x_hbm, table_hbm, out_hbm, idx, idx2, rows, tbuf, sems):
    _emb_body(x_hbm, table_hbm, out_hbm, idx, idx2, rows, tbuf, sems)


def kernel(x, table):
    x4 = (
        x.astype(jnp.int32)
        .T.reshape(TJ, 8, TI, SUB)
        .transpose(0, 2, 1, 3)
    )
    tp = _prep(table.T)
    out5 = _emb(x4, tp)
    return out5.transpose(2, 4, 0, 1, 3).reshape(NROWS, NCOLS, DMODEL)
